# baseline (speedup 1.0000x reference)
from functools import partial

import numpy as np
import jax
import jax.numpy as jnp
from jax import lax
from jax.experimental import pallas as pl
from jax.experimental.pallas import tpu as pltpu

_B, _T, _F = 16, 2048, 768
_RATIO = 0.3
_ZERO_RATIO = 0.8
_RANDOM_RATIO = 0.5
_TBLK = 256

_ROTS = ((13, 15, 26, 6), (17, 29, 16, 24))


def _np_threefry2x32(k1, k2, x0, x1):
    k1 = np.uint32(k1)
    k2 = np.uint32(k2)
    k3 = np.uint32(k1 ^ k2 ^ np.uint32(0x1BD11BDA))
    ks = (k1, k2, k3)
    x0 = (x0 + k1).astype(np.uint32)
    x1 = (x1 + k2).astype(np.uint32)
    for i in range(5):
        for r in _ROTS[i % 2]:
            x0 = (x0 + x1).astype(np.uint32)
            x1 = ((x1 << np.uint32(r)) | (x1 >> np.uint32(32 - r))).astype(np.uint32)
            x1 = x0 ^ x1
        x0 = (x0 + ks[(i + 1) % 3]).astype(np.uint32)
        x1 = (x1 + ks[(i + 2) % 3] + np.uint32(i + 1)).astype(np.uint32)
    return x0, x1


def _np_split_key(k1, k2, num):
    idx = np.arange(num, dtype=np.uint32)
    o0, o1 = _np_threefry2x32(k1, k2, np.zeros(num, np.uint32), idx)
    return list(zip(o0.tolist(), o1.tolist()))


_K_MASK, _K_ZERO, _K_RAND, _K_VALS = _np_split_key(0, 42, 4)

_o0, _o1 = _np_threefry2x32(
    _K_MASK[0], _K_MASK[1],
    np.zeros(_B * _F, np.uint32), np.arange(_B * _F, dtype=np.uint32))
_mbits = _o0 ^ _o1
_mfloat = (((_mbits >> np.uint32(9)) | np.uint32(0x3F800000))
           .view(np.float32) - np.float32(1.0))
_MASK2D = (_mfloat < np.float32(_RATIO)).reshape(_B, 1, _F)
_MASK2D_I32 = np.ascontiguousarray(_MASK2D.astype(np.int32))


def _i32(v):
    return np.uint32(v).astype(np.int32)


def _hash_bits(key, idx):
    k1 = _i32(key[0])
    k2 = _i32(key[1])
    k3 = _i32(np.uint32(key[0]) ^ np.uint32(key[1]) ^ np.uint32(0x1BD11BDA))
    ks = (k1, k2, k3)
    x0 = jnp.full(idx.shape, k1, jnp.int32)
    x1 = idx + k2
    for i in range(5):
        for r in _ROTS[i % 2]:
            x0 = x0 + x1
            x1 = lax.shift_left(x1, np.int32(r)) | lax.shift_right_logical(
                x1, np.int32(32 - r))
            x1 = x0 ^ x1
        x0 = x0 + ks[(i + 1) % 3]
        x1 = x1 + ks[(i + 2) % 3] + np.int32(i + 1)
    return x0 ^ x1


def _uniform_f32(bits):
    fb = lax.shift_right_logical(bits, np.int32(9)) | np.int32(0x3F800000)
    return lax.bitcast_convert_type(fb, jnp.float32) - jnp.float32(1.0)


def _tile_index(b, t):
    base = b * np.int32(_T * _F) + t * np.int32(_TBLK * _F)
    row = lax.broadcasted_iota(jnp.int32, (_TBLK, _F), 0)
    col = lax.broadcasted_iota(jnp.int32, (_TBLK, _F), 1)
    return base + row * np.int32(_F) + col


def _pass_a(x_ref, m_ref, zeroed_ref, mask_out_ref, zmask_ref, colmax_ref):
    b = pl.program_id(0)
    t = pl.program_id(1)
    x = x_ref[0]
    idx = _tile_index(b, t)
    zbits = _hash_bits(_K_ZERO, idx)
    z = _uniform_f32(zbits) < jnp.float32(_ZERO_RATIO)
    mrow = m_ref[0]
    zidx = z & (mrow > 0)
    zeroed = jnp.where(zidx, jnp.float32(0.0), x)
    zeroed_ref[0] = zeroed
    mask_out_ref[0] = jnp.broadcast_to(mrow, (_TBLK, _F))
    zmask_ref[0] = zidx.astype(jnp.int8)
    bmax = jnp.max(zeroed, axis=0, keepdims=True)

    @pl.when(t == 0)
    def _init():
        colmax_ref[0] = bmax

    @pl.when(t != 0)
    def _acc():
        colmax_ref[0] = jnp.maximum(colmax_ref[0], bmax)


def _pass_b(zeroed_ref, zmask_ref, m_ref, mx_ref, out_ref):
    b = pl.program_id(0)
    t = pl.program_id(1)
    idx = _tile_index(b, t)
    rbits = _hash_bits(_K_RAND, idx)
    r = _uniform_f32(rbits) < jnp.float32(_RANDOM_RATIO)
    v = _uniform_f32(_hash_bits(_K_VALS, idx))
    mrow = m_ref[0]
    ridx = r & (mrow > 0) & (zmask_ref[0] == 0)
    out_ref[0] = jnp.where(ridx, mx_ref[0, 0] * v, zeroed_ref[0])


def kernel(spikes):
    mask2d = jnp.asarray(_MASK2D_I32)
    grid = (_B, _T // _TBLK)
    xspec = pl.BlockSpec((1, _TBLK, _F), lambda b, t: (b, t, 0))
    mspec = pl.BlockSpec((1, 1, _F), lambda b, t: (b, 0, 0))
    cspec = pl.BlockSpec((1, 1, _F), lambda b, t: (b, 0, 0))

    zeroed, mask_out, zmask, colmax = pl.pallas_call(
        _pass_a,
        grid=grid,
        in_specs=[xspec, mspec],
        out_specs=[xspec, xspec, xspec, cspec],
        out_shape=[
            jax.ShapeDtypeStruct((_B, _T, _F), jnp.float32),
            jax.ShapeDtypeStruct((_B, _T, _F), jnp.int32),
            jax.ShapeDtypeStruct((_B, _T, _F), jnp.int8),
            jax.ShapeDtypeStruct((_B, 1, _F), jnp.float32),
        ],
    )(spikes, mask2d)

    mx = jnp.max(colmax).reshape(1, 1)

    out = pl.pallas_call(
        _pass_b,
        grid=grid,
        in_specs=[xspec, xspec, mspec,
                  pl.BlockSpec((1, 1), lambda b, t: (0, 0))],
        out_specs=xspec,
        out_shape=jax.ShapeDtypeStruct((_B, _T, _F), jnp.float32),
    )(zeroed, zmask, mask2d, mx)

    return out, mask_out.astype(jnp.int64)


# speedup vs baseline: 3.0497x; 3.0497x over previous
import numpy as np
import jax
import jax.numpy as jnp
from jax import lax
from jax.experimental import pallas as pl
from jax.experimental.pallas import tpu as pltpu

_B, _T, _F = 16, 2048, 768
_RATIO = 0.3
_ZERO_RATIO = 0.8
_RANDOM_RATIO = 0.5
_TBLK = 256
_JP = 256

_ROTS = ((13, 15, 26, 6), (17, 29, 16, 24))


def _np_threefry2x32(k1, k2, x0, x1):
    k1 = np.uint32(k1)
    k2 = np.uint32(k2)
    k3 = np.uint32(k1 ^ k2 ^ np.uint32(0x1BD11BDA))
    ks = (k1, k2, k3)
    x0 = (x0 + k1).astype(np.uint32)
    x1 = (x1 + k2).astype(np.uint32)
    for i in range(5):
        for r in _ROTS[i % 2]:
            x0 = (x0 + x1).astype(np.uint32)
            x1 = ((x1 << np.uint32(r)) | (x1 >> np.uint32(32 - r))).astype(np.uint32)
            x1 = x0 ^ x1
        x0 = (x0 + ks[(i + 1) % 3]).astype(np.uint32)
        x1 = (x1 + ks[(i + 2) % 3] + np.uint32(i + 1)).astype(np.uint32)
    return x0, x1


def _np_split_key(k1, k2, num):
    idx = np.arange(num, dtype=np.uint32)
    o0, o1 = _np_threefry2x32(k1, k2, np.zeros(num, np.uint32), idx)
    return list(zip(o0.tolist(), o1.tolist()))


_K_MASK, _K_ZERO, _K_RAND, _K_VALS = _np_split_key(0, 42, 4)

_o0, _o1 = _np_threefry2x32(
    _K_MASK[0], _K_MASK[1],
    np.zeros(_B * _F, np.uint32), np.arange(_B * _F, dtype=np.uint32))
_mbits = _o0 ^ _o1
_mfloat = (((_mbits >> np.uint32(9)) | np.uint32(0x3F800000))
           .view(np.float32) - np.float32(1.0))
_MASK2D = (_mfloat < np.float32(_RATIO)).reshape(_B, _F)
_MASK2D_I32 = np.ascontiguousarray(_MASK2D.astype(np.int32).reshape(_B, 1, _F))

_COLIDX = np.full((_B, 1, _JP), -1, dtype=np.int32)
for _b in range(_B):
    _nz = np.nonzero(_MASK2D[_b])[0].astype(np.int32)
    _COLIDX[_b, 0, : _nz.size] = _nz
_COLIDX_T = np.ascontiguousarray(_COLIDX.reshape(_B, _JP, 1))

_ZT = np.int32(int(np.ceil(np.float64(np.float32(_ZERO_RATIO)) * 2**23)))
_RT = np.int32(int(np.ceil(np.float64(np.float32(_RANDOM_RATIO)) * 2**23)))


def _i32(v):
    return np.uint32(v).astype(np.int32)


def _hash_bits(key, idx):
    k1 = _i32(key[0])
    k2 = _i32(key[1])
    k3 = _i32(np.uint32(key[0]) ^ np.uint32(key[1]) ^ np.uint32(0x1BD11BDA))
    ks = (k1, k2, k3)
    x0 = jnp.full(idx.shape, k1, jnp.int32)
    x1 = idx + k2
    for i in range(5):
        for r in _ROTS[i % 2]:
            x0 = x0 + x1
            x1 = lax.shift_left(x1, np.int32(r)) | lax.shift_right_logical(
                x1, np.int32(32 - r))
            x1 = x0 ^ x1
        x0 = x0 + ks[(i + 1) % 3]
        x1 = x1 + ks[(i + 2) % 3] + np.int32(i + 1)
    return x0 ^ x1


def _uniform_f32(bits):
    fb = lax.shift_right_logical(bits, np.int32(9)) | np.int32(0x3F800000)
    return lax.bitcast_convert_type(fb, jnp.float32) - jnp.float32(1.0)


def _compact_index(b, t, ci):
    base = b * np.int32(_T * _F) + t * np.int32(_TBLK * _F)
    row = lax.broadcasted_iota(jnp.int32, (_TBLK, _JP), 0)
    return base + row * np.int32(_F) + ci


def _pass_a(x_ref, ci_ref, m_ref, zm_ref, zmk_ref, maskout_ref,
            cmx_ref, cmz_ref):
    b = pl.program_id(0)
    t = pl.program_id(1)
    x = x_ref[0]
    ci = ci_ref[0]
    f_iota = lax.broadcasted_iota(jnp.int32, (_F, _JP), 0)
    p_gather = (f_iota == ci).astype(jnp.float32)
    xm = jnp.dot(x, p_gather, precision=lax.Precision.HIGHEST,
                 preferred_element_type=jnp.float32)
    idx = _compact_index(b, t, ci)
    zbits = _hash_bits(_K_ZERO, idx)
    z = lax.shift_right_logical(zbits, np.int32(9)) < _ZT
    zm = jnp.where(z, jnp.float32(0.0), xm)
    zm_ref[0] = zm
    zmk_ref[0] = z.astype(jnp.int8)
    maskout_ref[0] = jnp.broadcast_to(m_ref[0], (_TBLK, _F))
    bmax_x = jnp.max(x, axis=0, keepdims=True)
    bmax_z = jnp.max(zm, axis=0, keepdims=True)

    @pl.when(t == 0)
    def _init():
        cmx_ref[0] = bmax_x
        cmz_ref[0] = bmax_z

    @pl.when(t != 0)
    def _acc():
        cmx_ref[0] = jnp.maximum(cmx_ref[0], bmax_x)
        cmz_ref[0] = jnp.maximum(cmz_ref[0], bmax_z)


def _pass_b(x_ref, zm_ref, zmk_ref, ci_ref, cit_ref, m_ref, mx_ref, out_ref):
    b = pl.program_id(0)
    t = pl.program_id(1)
    ci = ci_ref[0]
    idx = _compact_index(b, t, ci)
    rbits = _hash_bits(_K_RAND, idx)
    r = lax.shift_right_logical(rbits, np.int32(9)) < _RT
    v = _uniform_f32(_hash_bits(_K_VALS, idx))
    ridx = r & (zmk_ref[0] == 0)
    outm = jnp.where(ridx, mx_ref[0, 0] * v, zm_ref[0])
    cit = cit_ref[0]
    f_iota = lax.broadcasted_iota(jnp.int32, (_JP, _F), 1)
    p_scatter = (f_iota == cit).astype(jnp.float32)
    scat = jnp.dot(outm, p_scatter, precision=lax.Precision.HIGHEST,
                   preferred_element_type=jnp.float32)
    out_ref[0] = jnp.where(m_ref[0] > 0, jnp.float32(0.0), x_ref[0]) + scat


def kernel(spikes):
    mask2d = jnp.asarray(_MASK2D_I32)
    colidx = jnp.asarray(_COLIDX)
    colidx_t = jnp.asarray(_COLIDX_T)
    grid = (_B, _T // _TBLK)
    xspec = pl.BlockSpec((1, _TBLK, _F), lambda b, t: (b, t, 0))
    cspec = pl.BlockSpec((1, _TBLK, _JP), lambda b, t: (b, t, 0))
    mspec = pl.BlockSpec((1, 1, _F), lambda b, t: (b, 0, 0))
    cispec = pl.BlockSpec((1, 1, _JP), lambda b, t: (b, 0, 0))
    citspec = pl.BlockSpec((1, _JP, 1), lambda b, t: (b, 0, 0))

    zm, zmk, mask_out, cmx, cmz = pl.pallas_call(
        _pass_a,
        grid=grid,
        in_specs=[xspec, cispec, mspec],
        out_specs=[cspec, cspec, xspec,
                   pl.BlockSpec((1, 1, _F), lambda b, t: (b, 0, 0)),
                   pl.BlockSpec((1, 1, _JP), lambda b, t: (b, 0, 0))],
        out_shape=[
            jax.ShapeDtypeStruct((_B, _T, _JP), jnp.float32),
            jax.ShapeDtypeStruct((_B, _T, _JP), jnp.int8),
            jax.ShapeDtypeStruct((_B, _T, _F), jnp.int32),
            jax.ShapeDtypeStruct((_B, 1, _F), jnp.float32),
            jax.ShapeDtypeStruct((_B, 1, _JP), jnp.float32),
        ],
    )(spikes, colidx, mask2d)

    unmasked_max = jnp.max(jnp.where(mask2d > 0, jnp.float32(-jnp.inf), cmx))
    mx = jnp.maximum(unmasked_max, jnp.max(cmz)).reshape(1, 1)

    out = pl.pallas_call(
        _pass_b,
        grid=grid,
        in_specs=[xspec, cspec, cspec, cispec, citspec, mspec,
                  pl.BlockSpec((1, 1), lambda b, t: (0, 0))],
        out_specs=xspec,
        out_shape=jax.ShapeDtypeStruct((_B, _T, _F), jnp.float32),
    )(spikes, zm, zmk, colidx, colidx_t, mask2d, mx)

    return out, mask_out.astype(jnp.int64)
